# Initial kernel scaffold; baseline (speedup 1.0000x reference)
import jax
import jax.numpy as jnp
from jax.experimental import pallas as pl


def kernel(feats, e_w, snorm_n, snorm_e, gt, maps, edge_index, params):
    raise NotImplementedError("write your pallas kernel here")


# speedup vs baseline: 1.0015x; 1.0015x over previous
import functools

import jax
import jax.numpy as jnp
from jax.experimental import pallas as pl
from jax.experimental.pallas import tpu as pltpu

N_NODES = 2048
HEADS = 2


def _mlp_dec_body(h_ref, w0_ref, b0_ref, w1_ref, b1_ref, w2_ref, b2_ref, o_ref):
    x = h_ref[...]
    x = jax.nn.relu(jnp.dot(x, w0_ref[...].T, preferred_element_type=jnp.float32) + b0_ref[...])
    x = jax.nn.relu(jnp.dot(x, w1_ref[...].T, preferred_element_type=jnp.float32) + b1_ref[...])
    o_ref[...] = jnp.dot(x, w2_ref[...].T, preferred_element_type=jnp.float32) + b2_ref[...]


def _mlp_dec(h, p):
    n, d = h.shape
    out_d = p["l2"]["W"].shape[0]
    tile = 256
    grid = (n // tile,)
    return pl.pallas_call(
        _mlp_dec_body,
        grid=grid,
        in_specs=[
            pl.BlockSpec((tile, d), lambda i: (i, 0)),
            pl.BlockSpec(p["l0"]["W"].shape, lambda i: (0, 0)),
            pl.BlockSpec((1, p["l0"]["b"].shape[0]), lambda i: (0, 0)),
            pl.BlockSpec(p["l1"]["W"].shape, lambda i: (0, 0)),
            pl.BlockSpec((1, p["l1"]["b"].shape[0]), lambda i: (0, 0)),
            pl.BlockSpec(p["l2"]["W"].shape, lambda i: (0, 0)),
            pl.BlockSpec((1, p["l2"]["b"].shape[0]), lambda i: (0, 0)),
        ],
        out_specs=pl.BlockSpec((tile, out_d), lambda i: (i, 0)),
        out_shape=jax.ShapeDtypeStruct((n, out_d), jnp.float32),
    )(h, p["l0"]["W"], p["l0"]["b"][None, :], p["l1"]["W"], p["l1"]["b"][None, :],
      p["l2"]["W"], p["l2"]["b"][None, :])


def _map_encoder(p, x):
    dn = ("NCHW", "OIHW", "NCHW")
    for name, s in (("c1", 2), ("c2", 2), ("c3", 1), ("c4", 1)):
        x = jax.lax.conv_general_dilated(x, p[name]["W"], (s, s), "VALID", dimension_numbers=dn)
        x = jax.nn.relu(x + p[name]["b"][None, :, None, None])
    x = x.reshape(x.shape[0], -1)
    return x @ p["fc"]["W"].T + p["fc"]["b"]


def _gat_layer(p, h, src, dst, w_e, snorm_n, att_ew, n):
    h_in = h
    h_s = h @ p["W_self"].T
    z = h @ p["W_func"].T
    z_src = jnp.take(z, src, axis=0)
    z_dst = jnp.take(z, dst, axis=0)
    if att_ew:
        cat = jnp.concatenate([z_src, z_dst, w_e], axis=-1)
    else:
        cat = jnp.concatenate([z_src, z_dst], axis=-1)
    e = jax.nn.leaky_relu(cat @ p["W_att"].T)
    e_max = jax.ops.segment_max(e, dst, num_segments=n)
    e_max = jnp.where(jnp.isfinite(e_max), e_max, 0.0)
    e_exp = jnp.exp(e - jnp.take(e_max, dst, axis=0))
    denom = jax.ops.segment_sum(e_exp, dst, num_segments=n)
    alpha = e_exp / (jnp.take(denom, dst, axis=0) + 1e-16)
    agg = jax.ops.segment_sum(alpha * z_src, dst, num_segments=n)
    deg = jax.ops.segment_sum(jnp.ones((dst.shape[0], 1), jnp.float32), dst, num_segments=n)
    h_new = jnp.where(deg > 0, h_s + agg, h_in)
    h_new = h_new * snorm_n
    h_res = h_in @ p["res"]["W"].T + p["res"]["b"]
    return jax.nn.relu(h_new + h_res)


def _gat_vae_fwd(p, h, src, dst, w_e1, e_w, snorm_n, att_ew, n):
    outs = [_gat_layer(hp, h, src, dst, w_e1, snorm_n, att_ew, n) for hp in p["gat1"]]
    h = jnp.concatenate(outs, axis=1)
    w_e2 = e_w @ p["emb_e"]["W"].T + p["emb_e"]["b"]
    outs2 = [_gat_layer(hp, h, src, dst, w_e2, snorm_n, att_ew, n) for hp in p["gat2"]]
    return jnp.concatenate(outs2, axis=1)


def kernel(feats, e_w, snorm_n, snorm_e, gt, maps, edge_index, params):
    n = feats.shape[0]
    feats = feats.reshape(n, -1)
    gt = gt.reshape(gt.shape[0], -1)
    src = edge_index[0]
    dst = edge_index[1]
    h_emb = feats @ params["embedding_h"]["W"].T + params["embedding_h"]["b"]
    w_e = e_w @ params["embedding_e"]["W"].T + params["embedding_e"]["b"]
    maps_emb = _map_encoder(params["map"], maps)
    h = jnp.concatenate([maps_emb, h_emb, gt], axis=-1)
    h = _gat_vae_fwd(params["gnn_enc"], h, src, dst, w_e, e_w, snorm_n, True, n)
    h = jnp.concatenate([h, gt], axis=-1)
    pe = params["encoder"]
    hh = jax.nn.leaky_relu(h @ pe["linear"]["W"].T + pe["linear"]["b"])
    log_var = hh @ pe["log_var"]["W"].T + pe["log_var"]["b"]
    mu = hh @ pe["mu"]["W"].T + pe["mu"]["b"]
    std = jnp.exp(0.5 * log_var)
    eps = jax.random.normal(jax.random.key(1234), std.shape, jnp.float32)
    z = mu + std * eps
    h_dec = jnp.concatenate([maps_emb, h_emb, z], axis=-1)
    h = _gat_vae_fwd(params["gnn_dec"], h_dec, src, dst, w_e, e_w, snorm_n, False, n)
    h = jnp.concatenate([h, z], axis=-1)
    recon_y = _mlp_dec(h, params["mlp_dec"])
    return (recon_y, mu, log_var)


# speedup vs baseline: 4.6725x; 4.6657x over previous
import functools

import jax
import jax.numpy as jnp
from jax import lax
from jax.experimental import pallas as pl
from jax.experimental.pallas import tpu as pltpu
from jax.experimental.pallas import tpu_sc as plsc

N = 2048
E = 32768
NS = 16
NCORE = 2
NHALF = N // NCORE
EPW = E // NS
VPW = EPW // 16
RPW = NHALF // NS
HROWS = 32
CHUNK = 2048
NEG = -1e30


def _edge_softmax_body(src_h, dst_h, s3_h, s1_h, s2_h, a_h, q_h,
                       src_sl, dst_sl, s3_sl, s1_v, s2_v, sc_sl,
                       c_loc, q_loc, tmp64, cg64, a_sl, csrc, cdst, cal,
                       sh_c, sh_g, sh_alpha):
    cid = lax.axis_index("c")
    sid = lax.axis_index("s")
    ebase = sid * EPW
    rbase = sid * RPW
    half_base = cid * NHALF

    pltpu.sync_copy(src_h.at[pl.ds(ebase, EPW)], src_sl)
    pltpu.sync_copy(dst_h.at[pl.ds(ebase, EPW)], dst_sl)
    pltpu.sync_copy(s3_h.at[pl.ds(ebase, EPW)], s3_sl)
    pltpu.sync_copy(s1_h, s1_v)
    pltpu.sync_copy(s2_h, s2_v)

    def memset(ref, n16, val):
        def body(i, _):
            ref[pl.ds(i * 16, 16)] = jnp.full((16,), val, jnp.float32)
            return 0
        lax.fori_loop(0, n16, body, 0)

    memset(c_loc, NHALF // 16, NEG)

    def p1(i, _):
        off = i * 16
        sl = src_sl[pl.ds(off, 16)]
        dl = dst_sl[pl.ds(off, 16)]
        s3l = s3_sl[pl.ds(off, 16)]
        x = plsc.load_gather(s1_v, [sl]) + plsc.load_gather(s2_v, [dl]) + s3l
        score = jnp.where(x >= 0.0, x, 0.01 * x)
        sc_sl[pl.ds(off, 16)] = score
        rel = dl - half_base
        m = (rel >= 0) & (rel < NHALF)
        relc = jnp.where(m, rel, 0)
        plsc.store_scatter(c_loc, [relc], score, mask=m)
        return 0
    lax.fori_loop(0, VPW, p1, 0)

    def combine(loc, init, is_max):
        plsc.subcore_barrier()
        pltpu.sync_copy(loc, sh_c.at[sid])
        plsc.subcore_barrier()
        for j in range(RPW // 16):
            cg64[pl.ds(j * 16, 16)] = jnp.full((16,), init, jnp.float32)
        for w in range(NS):
            pltpu.sync_copy(sh_c.at[w, pl.ds(rbase, RPW)], tmp64)
            for j in range(RPW // 16):
                o = pl.ds(j * 16, 16)
                if is_max:
                    cg64[o] = jnp.maximum(cg64[o], tmp64[o])
                else:
                    cg64[o] = cg64[o] + tmp64[o]
        pltpu.sync_copy(cg64, sh_g.at[pl.ds(rbase, RPW)])
        plsc.subcore_barrier()
        pltpu.sync_copy(sh_g, loc)
        plsc.subcore_barrier()

    combine(c_loc, NEG, True)

    memset(q_loc, NHALF // 16, 0.0)

    def p2(i, _):
        off = i * 16
        dl = dst_sl[pl.ds(off, 16)]
        score = sc_sl[pl.ds(off, 16)]
        rel = dl - half_base
        m = (rel >= 0) & (rel < NHALF)
        relc = jnp.where(m, rel, 0)
        c16 = plsc.load_gather(c_loc, [relc])
        p = jnp.exp(jnp.minimum(score - c16, 80.0))
        p = jnp.where(m, p, 0.0)
        sc_sl[pl.ds(off, 16)] = p
        plsc.addupdate_scatter(q_loc, [relc], p, mask=m)
        return 0
    lax.fori_loop(0, VPW, p2, 0)

    combine(q_loc, 0.0, False)
    pltpu.sync_copy(cg64, q_h.at[pl.ds(half_base + rbase, RPW)])

    def p2b(i, _):
        off = i * 16
        dl = dst_sl[pl.ds(off, 16)]
        p = sc_sl[pl.ds(off, 16)]
        rel = dl - half_base
        m = (rel >= 0) & (rel < NHALF)
        relc = jnp.where(m, rel, 0)
        q16 = plsc.load_gather(q_loc, [relc])
        alpha = p / (q16 + 1e-16)
        sc_sl[pl.ds(off, 16)] = jnp.where(m, alpha, 0.0)
        return 0
    lax.fori_loop(0, VPW, p2b, 0)
    pltpu.sync_copy(sc_sl, sh_alpha.at[pl.ds(ebase, EPW)])
    plsc.subcore_barrier()

    for half in range(RPW // HROWS):
        row0 = half_base + rbase + half * HROWS
        memset(a_sl, HROWS * N // 16, 0.0)
        for ck in range(E // CHUNK):
            pltpu.sync_copy(src_h.at[pl.ds(ck * CHUNK, CHUNK)], csrc)
            pltpu.sync_copy(dst_h.at[pl.ds(ck * CHUNK, CHUNK)], cdst)
            pltpu.sync_copy(sh_alpha.at[pl.ds(ck * CHUNK, CHUNK)], cal)

            def p3(i, _):
                off = i * 16
                dl = cdst[pl.ds(off, 16)]
                sl = csrc[pl.ds(off, 16)]
                al = cal[pl.ds(off, 16)]
                rel = dl - row0
                m = (rel >= 0) & (rel < HROWS)
                flat = jnp.where(m, rel, 0) * N + sl
                plsc.addupdate_scatter(a_sl, [flat], al, mask=m)
                return 0
            lax.fori_loop(0, CHUNK // 16, p3, 0)
        pltpu.sync_copy(a_sl, a_h.at[pl.ds(row0 * N, HROWS * N)])


@jax.jit
def _edge_softmax_sc(src, dst, s3, s1, s2):
    mesh = plsc.VectorSubcoreMesh(core_axis_name="c", subcore_axis_name="s")
    f = functools.partial(
        pl.kernel,
        out_type=(jax.ShapeDtypeStruct((N * N,), jnp.float32),
                  jax.ShapeDtypeStruct((N,), jnp.float32)),
        mesh=mesh,
        compiler_params=pltpu.CompilerParams(needs_layout_passes=False),
        scratch_types=[
            pltpu.VMEM((EPW,), jnp.int32),
            pltpu.VMEM((EPW,), jnp.int32),
            pltpu.VMEM((EPW,), jnp.float32),
            pltpu.VMEM((N,), jnp.float32),
            pltpu.VMEM((N,), jnp.float32),
            pltpu.VMEM((EPW,), jnp.float32),
            pltpu.VMEM((NHALF,), jnp.float32),
            pltpu.VMEM((NHALF,), jnp.float32),
            pltpu.VMEM((RPW,), jnp.float32),
            pltpu.VMEM((RPW,), jnp.float32),
            pltpu.VMEM((HROWS * N,), jnp.float32),
            pltpu.VMEM((CHUNK,), jnp.int32),
            pltpu.VMEM((CHUNK,), jnp.int32),
            pltpu.VMEM((CHUNK,), jnp.float32),
            pltpu.VMEM_SHARED((NS, NHALF), jnp.float32),
            pltpu.VMEM_SHARED((NHALF,), jnp.float32),
            pltpu.VMEM_SHARED((E,), jnp.float32),
        ],
    )(_edge_softmax_body)
    return f(src, dst, s3, s1, s2)


def _mlp_dec_body(h_ref, w0_ref, b0_ref, w1_ref, b1_ref, w2_ref, b2_ref, o_ref):
    x = h_ref[...]
    x = jax.nn.relu(jnp.dot(x, w0_ref[...].T, preferred_element_type=jnp.float32) + b0_ref[...])
    x = jax.nn.relu(jnp.dot(x, w1_ref[...].T, preferred_element_type=jnp.float32) + b1_ref[...])
    o_ref[...] = jnp.dot(x, w2_ref[...].T, preferred_element_type=jnp.float32) + b2_ref[...]


def _mlp_dec(h, p):
    n, d = h.shape
    out_d = p["l2"]["W"].shape[0]
    tile = 256
    grid = (n // tile,)
    return pl.pallas_call(
        _mlp_dec_body,
        grid=grid,
        in_specs=[
            pl.BlockSpec((tile, d), lambda i: (i, 0)),
            pl.BlockSpec(p["l0"]["W"].shape, lambda i: (0, 0)),
            pl.BlockSpec((1, p["l0"]["b"].shape[0]), lambda i: (0, 0)),
            pl.BlockSpec(p["l1"]["W"].shape, lambda i: (0, 0)),
            pl.BlockSpec((1, p["l1"]["b"].shape[0]), lambda i: (0, 0)),
            pl.BlockSpec(p["l2"]["W"].shape, lambda i: (0, 0)),
            pl.BlockSpec((1, p["l2"]["b"].shape[0]), lambda i: (0, 0)),
        ],
        out_specs=pl.BlockSpec((tile, out_d), lambda i: (i, 0)),
        out_shape=jax.ShapeDtypeStruct((n, out_d), jnp.float32),
    )(h, p["l0"]["W"], p["l0"]["b"][None, :], p["l1"]["W"], p["l1"]["b"][None, :],
      p["l2"]["W"], p["l2"]["b"][None, :])


def _map_encoder(p, x):
    dn = ("NCHW", "OIHW", "NCHW")
    for name, s in (("c1", 2), ("c2", 2), ("c3", 1), ("c4", 1)):
        x = jax.lax.conv_general_dilated(x, p[name]["W"], (s, s), "VALID", dimension_numbers=dn)
        x = jax.nn.relu(x + p[name]["b"][None, :, None, None])
    x = x.reshape(x.shape[0], -1)
    return x @ p["fc"]["W"].T + p["fc"]["b"]


def _gat_layer(p, h, src, dst, w_e, snorm_n, att_ew, n):
    h_s = h @ p["W_self"].T
    z = h @ p["W_func"].T
    d = z.shape[1]
    a = p["W_att"][0]
    s1 = z @ a[:d]
    s2 = z @ a[d:2 * d]
    if att_ew:
        s3 = w_e @ a[2 * d:]
    else:
        s3 = jnp.zeros((src.shape[0],), jnp.float32)
    a_flat, q = _edge_softmax_sc(src, dst, s3, s1, s2)
    agg = a_flat.reshape(n, n) @ z
    h_new = jnp.where(q[:, None] > 0, h_s + agg, h)
    h_new = h_new * snorm_n
    h_res = h @ p["res"]["W"].T + p["res"]["b"]
    return jax.nn.relu(h_new + h_res)


def _gat_vae_fwd(p, h, src, dst, w_e1, e_w, snorm_n, att_ew, n):
    outs = [_gat_layer(hp, h, src, dst, w_e1, snorm_n, att_ew, n) for hp in p["gat1"]]
    h = jnp.concatenate(outs, axis=1)
    w_e2 = e_w @ p["emb_e"]["W"].T + p["emb_e"]["b"]
    outs2 = [_gat_layer(hp, h, src, dst, w_e2, snorm_n, att_ew, n) for hp in p["gat2"]]
    return jnp.concatenate(outs2, axis=1)


def kernel(feats, e_w, snorm_n, snorm_e, gt, maps, edge_index, params):
    n = feats.shape[0]
    feats = feats.reshape(n, -1)
    gt = gt.reshape(gt.shape[0], -1)
    src = edge_index[0]
    dst = edge_index[1]
    h_emb = feats @ params["embedding_h"]["W"].T + params["embedding_h"]["b"]
    w_e = e_w @ params["embedding_e"]["W"].T + params["embedding_e"]["b"]
    maps_emb = _map_encoder(params["map"], maps)
    h = jnp.concatenate([maps_emb, h_emb, gt], axis=-1)
    h = _gat_vae_fwd(params["gnn_enc"], h, src, dst, w_e, e_w, snorm_n, True, n)
    h = jnp.concatenate([h, gt], axis=-1)
    pe = params["encoder"]
    hh = jax.nn.leaky_relu(h @ pe["linear"]["W"].T + pe["linear"]["b"])
    log_var = hh @ pe["log_var"]["W"].T + pe["log_var"]["b"]
    mu = hh @ pe["mu"]["W"].T + pe["mu"]["b"]
    std = jnp.exp(0.5 * log_var)
    eps = jax.random.normal(jax.random.key(1234), std.shape, jnp.float32)
    z = mu + std * eps
    h_dec = jnp.concatenate([maps_emb, h_emb, z], axis=-1)
    h = _gat_vae_fwd(params["gnn_dec"], h_dec, src, dst, w_e, e_w, snorm_n, False, n)
    h = jnp.concatenate([h, z], axis=-1)
    recon_y = _mlp_dec(h, params["mlp_dec"])
    return (recon_y, mu, log_var)


# speedup vs baseline: 5.4424x; 1.1648x over previous
import functools

import jax
import jax.numpy as jnp
from jax import lax
from jax.experimental import pallas as pl
from jax.experimental.pallas import tpu as pltpu
from jax.experimental.pallas import tpu_sc as plsc

N = 2048
E = 32768
NS = 16
NCORE = 2
NHALF = N // NCORE
EPW = E // NS
VPW = EPW // 16
RPW = NHALF // NS
HROWS = 32
CHUNK = 4096
NEG = -1e30


def _edge_softmax_body(src_h, dst_h, s3_h, s1_h, s2_h, a_h, q_h,
                       src_sl, dst_sl, s3_sl, s1_v, s2_v, sc_sl,
                       c_loc, q_loc, tmp2, cg64, a_sl, csrc, cdst, cal,
                       sh_c, sh_g, sh_alpha, sh_zero):
    cid = lax.axis_index("c")
    sid = lax.axis_index("s")
    ebase = sid * EPW
    rbase = sid * RPW
    half_base = cid * NHALF

    pltpu.sync_copy(src_h.at[pl.ds(ebase, EPW)], src_sl)
    pltpu.sync_copy(dst_h.at[pl.ds(ebase, EPW)], dst_sl)
    pltpu.sync_copy(s3_h.at[pl.ds(ebase, EPW)], s3_sl)
    pltpu.sync_copy(s1_h, s1_v)
    pltpu.sync_copy(s2_h, s2_v)

    def memset(ref, n16, val):
        def body(i, _):
            ref[pl.ds(i * 16, 16)] = jnp.full((16,), val, jnp.float32)
            return 0
        lax.fori_loop(0, n16, body, 0)

    stripe = HROWS * N // NS
    def zbody(i, _):
        a_sl[pl.ds(i * 16, 16)] = jnp.zeros((16,), jnp.float32)
        return 0
    lax.fori_loop(0, stripe // 16, zbody, 0)
    pltpu.sync_copy(a_sl.at[pl.ds(0, stripe)], sh_zero.at[pl.ds(sid * stripe, stripe)])

    memset(c_loc, NHALF // 16, NEG)

    def p1(i, _):
        off = i * 16
        sl = src_sl[pl.ds(off, 16)]
        dl = dst_sl[pl.ds(off, 16)]
        s3l = s3_sl[pl.ds(off, 16)]
        x = plsc.load_gather(s1_v, [sl]) + plsc.load_gather(s2_v, [dl]) + s3l
        score = jnp.where(x >= 0.0, x, 0.01 * x)
        sc_sl[pl.ds(off, 16)] = score
        rel = dl - half_base
        m = (rel >= 0) & (rel < NHALF)
        relc = jnp.where(m, rel, 0)
        plsc.store_scatter(c_loc, [relc], score, mask=m)
        return 0
    lax.fori_loop(0, VPW, p1, 0)

    def combine(loc, init, is_max):
        plsc.subcore_barrier()
        pltpu.sync_copy(loc, sh_c.at[sid])
        plsc.subcore_barrier()
        pltpu.sync_copy(sh_c, tmp2)
        for j in range(RPW // 16):
            cg64[pl.ds(j * 16, 16)] = jnp.full((16,), init, jnp.float32)
        for w in range(NS):
            for j in range(RPW // 16):
                o = pl.ds(j * 16, 16)
                oi = pl.ds(rbase + j * 16, 16)
                if is_max:
                    cg64[o] = jnp.maximum(cg64[o], tmp2[w, oi])
                else:
                    cg64[o] = cg64[o] + tmp2[w, oi]
        pltpu.sync_copy(cg64, sh_g.at[pl.ds(rbase, RPW)])
        plsc.subcore_barrier()
        pltpu.sync_copy(sh_g, loc)
        plsc.subcore_barrier()

    combine(c_loc, NEG, True)

    memset(q_loc, NHALF // 16, 0.0)

    def p2(i, _):
        off = i * 16
        dl = dst_sl[pl.ds(off, 16)]
        score = sc_sl[pl.ds(off, 16)]
        rel = dl - half_base
        m = (rel >= 0) & (rel < NHALF)
        relc = jnp.where(m, rel, 0)
        c16 = plsc.load_gather(c_loc, [relc])
        p = jnp.exp(jnp.minimum(score - c16, 80.0))
        p = jnp.where(m, p, 0.0)
        sc_sl[pl.ds(off, 16)] = p
        plsc.addupdate_scatter(q_loc, [relc], p, mask=m)
        return 0
    lax.fori_loop(0, VPW, p2, 0)

    combine(q_loc, 0.0, False)
    pltpu.sync_copy(cg64, q_h.at[pl.ds(half_base + rbase, RPW)])

    def p2b(i, _):
        off = i * 16
        dl = dst_sl[pl.ds(off, 16)]
        p = sc_sl[pl.ds(off, 16)]
        rel = dl - half_base
        m = (rel >= 0) & (rel < NHALF)
        relc = jnp.where(m, rel, 0)
        q16 = plsc.load_gather(q_loc, [relc])
        alpha = p / (q16 + 1e-16)
        sc_sl[pl.ds(off, 16)] = jnp.where(m, alpha, 0.0)
        return 0
    lax.fori_loop(0, VPW, p2b, 0)
    pltpu.sync_copy(sc_sl, sh_alpha.at[pl.ds(ebase, EPW)])
    plsc.subcore_barrier()

    ncks = E // CHUNK
    for half in range(RPW // HROWS):
        row0 = half_base + rbase + half * HROWS
        pltpu.sync_copy(sh_zero, a_sl)
        for ck in range(ncks):
            o = pl.ds(ck * CHUNK, CHUNK)
            pltpu.sync_copy(src_h.at[o], csrc)
            pltpu.sync_copy(dst_h.at[o], cdst)
            pltpu.sync_copy(sh_alpha.at[o], cal)

            def p3(i, _):
                off = pl.ds(i * 16, 16)
                dl = cdst[off]
                sl = csrc[off]
                al = cal[off]
                rel = dl - row0
                m = (rel >= 0) & (rel < HROWS)
                flat = jnp.where(m, rel, 0) * N + sl
                plsc.addupdate_scatter(a_sl, [flat], al, mask=m)
                return 0
            lax.fori_loop(0, CHUNK // 16, p3, 0)
        pltpu.sync_copy(a_sl, a_h.at[pl.ds(row0 * N, HROWS * N)])


@jax.jit
def _edge_softmax_sc(src, dst, s3, s1, s2):
    mesh = plsc.VectorSubcoreMesh(core_axis_name="c", subcore_axis_name="s")
    f = functools.partial(
        pl.kernel,
        out_type=(jax.ShapeDtypeStruct((N * N,), jnp.float32),
                  jax.ShapeDtypeStruct((N,), jnp.float32)),
        mesh=mesh,
        compiler_params=pltpu.CompilerParams(needs_layout_passes=False),
        scratch_types=[
            pltpu.VMEM((EPW,), jnp.int32),
            pltpu.VMEM((EPW,), jnp.int32),
            pltpu.VMEM((EPW,), jnp.float32),
            pltpu.VMEM((N,), jnp.float32),
            pltpu.VMEM((N,), jnp.float32),
            pltpu.VMEM((EPW,), jnp.float32),
            pltpu.VMEM((NHALF,), jnp.float32),
            pltpu.VMEM((NHALF,), jnp.float32),
            pltpu.VMEM((NS, NHALF), jnp.float32),
            pltpu.VMEM((RPW,), jnp.float32),
            pltpu.VMEM((HROWS * N,), jnp.float32),
            pltpu.VMEM((CHUNK,), jnp.int32),
            pltpu.VMEM((CHUNK,), jnp.int32),
            pltpu.VMEM((CHUNK,), jnp.float32),
            pltpu.VMEM_SHARED((NS, NHALF), jnp.float32),
            pltpu.VMEM_SHARED((NHALF,), jnp.float32),
            pltpu.VMEM_SHARED((E,), jnp.float32),
            pltpu.VMEM_SHARED((HROWS * N,), jnp.float32),
        ],
    )(_edge_softmax_body)
    return f(src, dst, s3, s1, s2)


def _mlp_dec_body(h_ref, w0_ref, b0_ref, w1_ref, b1_ref, w2_ref, b2_ref, o_ref):
    x = h_ref[...]
    x = jax.nn.relu(jnp.dot(x, w0_ref[...].T, preferred_element_type=jnp.float32) + b0_ref[...])
    x = jax.nn.relu(jnp.dot(x, w1_ref[...].T, preferred_element_type=jnp.float32) + b1_ref[...])
    o_ref[...] = jnp.dot(x, w2_ref[...].T, preferred_element_type=jnp.float32) + b2_ref[...]


def _mlp_dec(h, p):
    n, d = h.shape
    out_d = p["l2"]["W"].shape[0]
    tile = 256
    grid = (n // tile,)
    return pl.pallas_call(
        _mlp_dec_body,
        grid=grid,
        in_specs=[
            pl.BlockSpec((tile, d), lambda i: (i, 0)),
            pl.BlockSpec(p["l0"]["W"].shape, lambda i: (0, 0)),
            pl.BlockSpec((1, p["l0"]["b"].shape[0]), lambda i: (0, 0)),
            pl.BlockSpec(p["l1"]["W"].shape, lambda i: (0, 0)),
            pl.BlockSpec((1, p["l1"]["b"].shape[0]), lambda i: (0, 0)),
            pl.BlockSpec(p["l2"]["W"].shape, lambda i: (0, 0)),
            pl.BlockSpec((1, p["l2"]["b"].shape[0]), lambda i: (0, 0)),
        ],
        out_specs=pl.BlockSpec((tile, out_d), lambda i: (i, 0)),
        out_shape=jax.ShapeDtypeStruct((n, out_d), jnp.float32),
    )(h, p["l0"]["W"], p["l0"]["b"][None, :], p["l1"]["W"], p["l1"]["b"][None, :],
      p["l2"]["W"], p["l2"]["b"][None, :])


def _map_encoder(p, x):
    dn = ("NCHW", "OIHW", "NCHW")
    for name, s in (("c1", 2), ("c2", 2), ("c3", 1), ("c4", 1)):
        x = jax.lax.conv_general_dilated(x, p[name]["W"], (s, s), "VALID", dimension_numbers=dn)
        x = jax.nn.relu(x + p[name]["b"][None, :, None, None])
    x = x.reshape(x.shape[0], -1)
    return x @ p["fc"]["W"].T + p["fc"]["b"]


def _gat_layer(p, h, src, dst, w_e, snorm_n, att_ew, n):
    h_s = h @ p["W_self"].T
    z = h @ p["W_func"].T
    d = z.shape[1]
    a = p["W_att"][0]
    s1 = z @ a[:d]
    s2 = z @ a[d:2 * d]
    if att_ew:
        s3 = w_e @ a[2 * d:]
    else:
        s3 = jnp.zeros((src.shape[0],), jnp.float32)
    a_flat, q = _edge_softmax_sc(src, dst, s3, s1, s2)
    agg = a_flat.reshape(n, n) @ z
    h_new = jnp.where(q[:, None] > 0, h_s + agg, h)
    h_new = h_new * snorm_n
    h_res = h @ p["res"]["W"].T + p["res"]["b"]
    return jax.nn.relu(h_new + h_res)


def _gat_vae_fwd(p, h, src, dst, w_e1, e_w, snorm_n, att_ew, n):
    outs = [_gat_layer(hp, h, src, dst, w_e1, snorm_n, att_ew, n) for hp in p["gat1"]]
    h = jnp.concatenate(outs, axis=1)
    w_e2 = e_w @ p["emb_e"]["W"].T + p["emb_e"]["b"]
    outs2 = [_gat_layer(hp, h, src, dst, w_e2, snorm_n, att_ew, n) for hp in p["gat2"]]
    return jnp.concatenate(outs2, axis=1)


def kernel(feats, e_w, snorm_n, snorm_e, gt, maps, edge_index, params):
    n = feats.shape[0]
    feats = feats.reshape(n, -1)
    gt = gt.reshape(gt.shape[0], -1)
    src = edge_index[0]
    dst = edge_index[1]
    h_emb = feats @ params["embedding_h"]["W"].T + params["embedding_h"]["b"]
    w_e = e_w @ params["embedding_e"]["W"].T + params["embedding_e"]["b"]
    maps_emb = _map_encoder(params["map"], maps)
    h = jnp.concatenate([maps_emb, h_emb, gt], axis=-1)
    h = _gat_vae_fwd(params["gnn_enc"], h, src, dst, w_e, e_w, snorm_n, True, n)
    h = jnp.concatenate([h, gt], axis=-1)
    pe = params["encoder"]
    hh = jax.nn.leaky_relu(h @ pe["linear"]["W"].T + pe["linear"]["b"])
    log_var = hh @ pe["log_var"]["W"].T + pe["log_var"]["b"]
    mu = hh @ pe["mu"]["W"].T + pe["mu"]["b"]
    std = jnp.exp(0.5 * log_var)
    eps = jax.random.normal(jax.random.key(1234), std.shape, jnp.float32)
    z = mu + std * eps
    h_dec = jnp.concatenate([maps_emb, h_emb, z], axis=-1)
    h = _gat_vae_fwd(params["gnn_dec"], h_dec, src, dst, w_e, e_w, snorm_n, False, n)
    h = jnp.concatenate([h, z], axis=-1)
    recon_y = _mlp_dec(h, params["mlp_dec"])
    return (recon_y, mu, log_var)
